# Initial kernel scaffold; baseline (speedup 1.0000x reference)
import jax
import jax.numpy as jnp
from jax.experimental import pallas as pl


def kernel(item_seq, item_seq_len, target_id, time_seq, time_interval_seq, target_time, q_emb, p_emb, q_time_w, p_time_w, w_time):
    raise NotImplementedError("write your pallas kernel here")


# speedup vs baseline: 4.5799x; 4.5799x over previous
import jax
import jax.numpy as jnp
from jax import lax
from jax.experimental import pallas as pl
from jax.experimental.pallas import tpu as pltpu
from jax.experimental.pallas import tpu_sc as plsc

B, L, V, D, K = 1024, 50, 100000, 64, 100
CH = 512
NCH = (V + CH - 1) // CH
VP = NCH * CH
NEG = -1e30
GRAN = 24.0 * 30.0 * 6.0

NCS = 2
NSS = 16
NW = NCS * NSS
RPW = B // NW
LP = 64
CMP = 256
CAP = 1024
GNE = 128
BIGI = 1 << 30


def _gumbel_const():
    u = jax.random.uniform(jax.random.key(1), (B, V), minval=1e-9, maxval=1.0)
    g = -jnp.log(-jnp.log(u))
    return jnp.pad(g, ((0, 0), (0, VP - V)))


def _lanes():
    return lax.iota(jnp.int32, 16)


def _bcast_i32(x):
    return jnp.full((16,), x, jnp.int32)


def _scatter1(ref, pos, val):
    plsc.store_scatter(ref, [_bcast_i32(pos)],
                       jnp.full((16,), val, ref.dtype), mask=_lanes() == 0)


def _gather1(ref, pos):
    return jnp.max(plsc.load_gather(ref, [_bcast_i32(pos)]))


def _encode_sc_body(iseq_hbm, ts_hbm, ti_hbm, len_hbm,
                    qemb_hbm, pemb_hbm,
                    pq_hbm, pp_hbm, sm_hbm,
                    isv, tsv, tiv, lenv, eq, ep, hqb, hpb, smb, sem1, sem2):
    wid = lax.axis_index("s") * NCS + lax.axis_index("c")
    b0 = wid * RPW
    zvec = jnp.zeros((16,), jnp.float32)

    pltpu.sync_copy(iseq_hbm.at[pl.ds(b0 * LP, RPW * LP)], isv)
    pltpu.sync_copy(ts_hbm.at[pl.ds(b0 * LP, RPW * LP)], tsv)
    pltpu.sync_copy(ti_hbm.at[pl.ds(b0 * LP, RPW * LP)], tiv)
    pltpu.sync_copy(len_hbm.at[pl.ds(b0, RPW)], lenv.at[pl.ds(0, RPW)])
    for k in range(RPW * 8 // 16):
        smb[pl.ds(k * 16, 16)] = zvec

    def row(r, _):
        o = r * LP
        h1 = pltpu.async_copy(qemb_hbm.at[isv.at[pl.ds(o, LP)]], eq, sem1)
        h2 = pltpu.async_copy(pemb_hbm.at[isv.at[pl.ds(o, LP)]], ep, sem2)

        lenb = _gather1(lenv, r)
        invv = jnp.ones((16,), jnp.float32) / jnp.full((16,), lenb, jnp.int32).astype(jnp.float32)
        tmean = sum(jnp.sum(tsv[pl.ds(o + k * 16, 16)]) for k in range(4)) * (1.0 / L)
        imean = sum(jnp.sum(tiv[pl.ds(o + k * 16, 16)]) for k in range(4)) * (1.0 / L)
        lt_s = _gather1(tsv, o + lenb - 1)

        def pool(rows_ref, out_ref):
            def al(l, accs):
                return tuple(accs[k] + rows_ref[l, pl.ds(k * 16, 16)]
                             for k in range(4))
            accs = lax.fori_loop(0, lenb, al, (zvec, zvec, zvec, zvec))
            for k in range(4):
                out_ref[pl.ds(o + k * 16, 16)] = accs[k] * invv

        h1.wait()
        pool(eq, hqb)
        h2.wait()
        pool(ep, hpb)

        _scatter1(smb, r * 8 + 0, tmean)
        _scatter1(smb, r * 8 + 1, imean)
        _scatter1(smb, r * 8 + 2, lt_s)
        return 0

    lax.fori_loop(0, RPW, row, 0)
    pltpu.sync_copy(hqb, pq_hbm.at[pl.ds(b0 * D, RPW * D)])
    pltpu.sync_copy(hpb, pp_hbm.at[pl.ds(b0 * D, RPW * D)])
    pltpu.sync_copy(smb, sm_hbm.at[pl.ds(b0 * 8, RPW * 8)])


def _encode_sc_call(iseq_flat, ts_flat, ti_flat, lens, q_emb, p_emb):
    mesh = plsc.VectorSubcoreMesh(core_axis_name="c", subcore_axis_name="s")
    fn = pl.kernel(
        _encode_sc_body,
        mesh=mesh,
        compiler_params=pltpu.CompilerParams(needs_layout_passes=False),
        out_type=[
            jax.ShapeDtypeStruct((B * D,), jnp.float32),
            jax.ShapeDtypeStruct((B * D,), jnp.float32),
            jax.ShapeDtypeStruct((B * 8,), jnp.float32),
        ],
        scratch_types=[
            pltpu.VMEM((RPW * LP,), jnp.int32),
            pltpu.VMEM((RPW * LP,), jnp.float32),
            pltpu.VMEM((RPW * LP,), jnp.float32),
            pltpu.VMEM((128,), jnp.int32),
            pltpu.VMEM((LP, 128), jnp.float32),
            pltpu.VMEM((LP, 128), jnp.float32),
            pltpu.VMEM((RPW * D,), jnp.float32),
            pltpu.VMEM((RPW * D,), jnp.float32),
            pltpu.VMEM((RPW * 8,), jnp.float32),
            pltpu.SemaphoreType.DMA,
            pltpu.SemaphoreType.DMA,
        ],
    )
    return fn(iseq_flat, ts_flat, ti_flat, lens, q_emb, p_emb)


def _scores_body(pq_ref, pp_ref, tm_ref, im_ref, tt_ref, qtw_ref, ptw_ref,
                 qe_ref, pe_ref, g_ref, tgt_ref,
                 sc_ref, cm_ref, tl_ref, ap0_ref, hp_ref):
    j = pl.program_id(0)
    h_q = (pq_ref[...] + tm_ref[...] * qtw_ref[0:1, :]
           + im_ref[...] * qtw_ref[1:2, :] + tt_ref[...] * qtw_ref[2:3, :])
    h_p = (pp_ref[...] + tm_ref[...] * ptw_ref[0:1, :]
           + im_ref[...] * ptw_ref[1:2, :] + tt_ref[...] * ptw_ref[2:3, :])
    logits = lax.dot_general(h_q, qe_ref[...], (((1,), (1,)), ((), ())),
                             preferred_element_type=jnp.float32)
    plog = lax.dot_general(h_p, pe_ref[...], (((1,), (1,)), ((), ())),
                           preferred_element_type=jnp.float32)
    col = j * CH + lax.broadcasted_iota(jnp.int32, (B, CH), 1)
    is_t = col == tgt_ref[...]
    valid = (col < V) & jnp.logical_not(is_t)
    score = jnp.where(valid, logits + g_ref[...], NEG)
    sc_ref[...] = score
    cm_ref[...] = jnp.max(score, axis=1, keepdims=True)[None]
    tpart = jnp.sum(jnp.where(is_t, logits, 0.0), axis=1, keepdims=True)
    apart = jnp.sum(jnp.where(is_t, plog, 0.0), axis=1, keepdims=True)

    @pl.when(j == 0)
    def _():
        tl_ref[...] = tpart
        ap0_ref[...] = apart
        hp_ref[...] = h_p

    @pl.when(j != 0)
    def _():
        tl_ref[...] += tpart
        ap0_ref[...] += apart


def _scores_call(pooled_q, pooled_p, tm, im, tt, qtw, ptw,
                 q_emb_pad, p_emb_pad, gumbel, target_id):
    return pl.pallas_call(
        _scores_body,
        grid=(NCH,),
        in_specs=[
            pl.BlockSpec((B, D), lambda j: (0, 0)),
            pl.BlockSpec((B, D), lambda j: (0, 0)),
            pl.BlockSpec((B, 1), lambda j: (0, 0)),
            pl.BlockSpec((B, 1), lambda j: (0, 0)),
            pl.BlockSpec((B, 1), lambda j: (0, 0)),
            pl.BlockSpec((3, D), lambda j: (0, 0)),
            pl.BlockSpec((3, D), lambda j: (0, 0)),
            pl.BlockSpec((CH, D), lambda j: (j, 0)),
            pl.BlockSpec((CH, D), lambda j: (j, 0)),
            pl.BlockSpec((B, CH), lambda j: (0, j)),
            pl.BlockSpec((B, 1), lambda j: (0, 0)),
        ],
        out_specs=[
            pl.BlockSpec((B, CH), lambda j: (0, j)),
            pl.BlockSpec((1, B, 1), lambda j: (j, 0, 0)),
            pl.BlockSpec((B, 1), lambda j: (0, 0)),
            pl.BlockSpec((B, 1), lambda j: (0, 0)),
            pl.BlockSpec((B, D), lambda j: (0, 0)),
        ],
        out_shape=[
            jax.ShapeDtypeStruct((B, VP), jnp.float32),
            jax.ShapeDtypeStruct((NCH, B, 1), jnp.float32),
            jax.ShapeDtypeStruct((B, 1), jnp.float32),
            jax.ShapeDtypeStruct((B, 1), jnp.float32),
            jax.ShapeDtypeStruct((B, D), jnp.float32),
        ],
    )(pooled_q, pooled_p, tm, im, tt, qtw, ptw,
      q_emb_pad, p_emb_pad, gumbel, target_id)


def _extract_top(n_out, keys_ref, nv, record):
    lanes = _lanes()

    def one(i, _):
        def mx(k, m):
            return jnp.maximum(m, keys_ref[pl.ds(k * 16, 16)])
        mv = lax.fori_loop(0, nv, mx, jnp.full((16,), NEG, jnp.float32))
        m = jnp.max(mv)

        def fp(k, p):
            v = keys_ref[pl.ds(k * 16, 16)]
            cand = jnp.where(v == m, lanes + k * 16, BIGI)
            return jnp.minimum(p, jnp.min(cand))
        p = lax.fori_loop(0, nv, fp, jnp.int32(BIGI))
        record(i, m, p)
        plsc.store_scatter(keys_ref, [_bcast_i32(p)],
                           jnp.full((16,), NEG, jnp.float32), mask=lanes == 0)
        return m

    return lax.fori_loop(0, n_out, one, jnp.float32(0.0))


def _select_body(cm_hbm, sc_hbm, gum_hbm, hp_hbm, pemb_hbm,
                 nl_hbm, ap_hbm,
                 cmv, gidx, actb, ck, cix, topv, topi, gidx2, gv,
                 prow, hpv, nlb, apb, sem1, sem2):
    wid = lax.axis_index("s") * NCS + lax.axis_index("c")
    b0 = wid * RPW
    lanes = _lanes()
    zvec_i = jnp.zeros((16,), jnp.int32)
    nvec_f = jnp.full((16,), NEG, jnp.float32)

    pltpu.sync_copy(hp_hbm.at[pl.ds(b0 * D, RPW * D)], hpv)

    def row(r, _):
        b = b0 + r
        pltpu.sync_copy(cm_hbm.at[pl.ds(b * CMP, CMP)], cmv)

        for k in range(GNE // 16):
            gidx[pl.ds(k * 16, 16)] = b * NCH + lanes + k * 16

        def rec_chunk(i, m, p):
            _scatter1(gidx, i, b * NCH + p)
        tau = _extract_top(K, cmv, CMP // 16, rec_chunk)

        pltpu.async_copy(sc_hbm.at[gidx], actb, sem1).wait()

        for k in range(CAP // 16):
            ck[pl.ds(k * 16, 16)] = nvec_f
            cix[pl.ds(k * 16, 16)] = zvec_i

        def chunk(c, off):
            cidv = plsc.load_gather(gidx, [_bcast_i32(c)]) - b * NCH
            colbase = cidv * CH
            for j in range(CH // 16):
                v = actb[c, pl.ds(j * 16, 16)]
                msk = v >= tau
                n = jnp.max(plsc.all_reduce_population_count(msk))

                @pl.when(n > 0)
                def _():
                    plsc.store_compressed(ck.at[pl.ds(off, 16)], v, mask=msk)
                    plsc.store_compressed(cix.at[pl.ds(off, 16)],
                                          colbase + j * 16 + lanes, mask=msk)
                off = jnp.minimum(off + n, CAP - 16)
            return off
        ncand = lax.fori_loop(0, K, chunk, jnp.int32(0))
        nv = (ncand + 15) >> 4

        for k in range(GNE // 16):
            topi[pl.ds(k * 16, 16)] = zvec_i
            topv[pl.ds(k * 16, 16)] = nvec_f

        def rec_cand(i, m, p):
            _scatter1(topv, i, m)
            _scatter1(topi, i, _gather1(cix, p))
        _extract_top(K, ck, nv, rec_cand)

        for k in range(GNE // 16):
            gidx2[pl.ds(k * 16, 16)] = topi[pl.ds(k * 16, 16)] + b * VP
        pltpu.async_copy(gum_hbm.at[gidx2], gv, sem1).wait()
        for k in range(GNE // 16):
            nlb[pl.ds(r * GNE + k * 16, 16)] = (topv[pl.ds(k * 16, 16)]
                                                - gv[pl.ds(k * 16, 16)])

        pltpu.async_copy(pemb_hbm.at[topi], prow, sem2).wait()
        hpk = tuple(hpv[pl.ds(r * D + k * 16, 16)] for k in range(4))

        def pdot(k2, _):
            dot = sum(jnp.sum(prow[k2, pl.ds(k * 16, 16)] * hpk[k])
                      for k in range(4))
            _scatter1(apb, r * GNE + k2, dot)
            return 0
        lax.fori_loop(0, K, pdot, 0)
        return 0

    lax.fori_loop(0, RPW, row, 0)
    pltpu.sync_copy(nlb, nl_hbm.at[pl.ds(b0 * GNE, RPW * GNE)])
    pltpu.sync_copy(apb, ap_hbm.at[pl.ds(b0 * GNE, RPW * GNE)])


def _select_call(cm_flat, sc2d, gum_flat, hp_flat, p_emb):
    mesh = plsc.VectorSubcoreMesh(core_axis_name="c", subcore_axis_name="s")
    fn = pl.kernel(
        _select_body,
        mesh=mesh,
        compiler_params=pltpu.CompilerParams(needs_layout_passes=False),
        out_type=[
            jax.ShapeDtypeStruct((B * GNE,), jnp.float32),
            jax.ShapeDtypeStruct((B * GNE,), jnp.float32),
        ],
        scratch_types=[
            pltpu.VMEM((CMP,), jnp.float32),
            pltpu.VMEM((GNE,), jnp.int32),
            pltpu.VMEM((GNE, CH), jnp.float32),
            pltpu.VMEM((CAP,), jnp.float32),
            pltpu.VMEM((CAP,), jnp.int32),
            pltpu.VMEM((GNE,), jnp.float32),
            pltpu.VMEM((GNE,), jnp.int32),
            pltpu.VMEM((GNE,), jnp.int32),
            pltpu.VMEM((GNE,), jnp.float32),
            pltpu.VMEM((GNE, 128), jnp.float32),
            pltpu.VMEM((RPW * D,), jnp.float32),
            pltpu.VMEM((RPW * GNE,), jnp.float32),
            pltpu.VMEM((RPW * GNE,), jnp.float32),
            pltpu.SemaphoreType.DMA,
            pltpu.SemaphoreType.DMA,
        ],
    )
    return fn(cm_flat, sc2d, gum_flat, hp_flat, p_emb)


_BD = 256


def _epi_body(nl_ref, ap_ref, hp_ref, wt_ref, lt_ref, tt_ref,
              out_ref, acc_ref):
    i = pl.program_id(0)

    @pl.when(i == 0)
    def _():
        acc_ref[0] = 0.0
        acc_ref[1] = 0.0

    nl = nl_ref[...]
    m = jnp.max(nl, axis=1, keepdims=True)
    e = jnp.exp(nl - m)
    noise_p = e / jnp.sum(e, axis=1, keepdims=True)

    ap = ap_ref[...]
    m2 = jnp.max(ap, axis=1, keepdims=True)
    e2 = jnp.exp(ap - m2)
    act_p = e2 / jnp.sum(e2, axis=1, keepdims=True)

    deno = K * noise_p + act_p + 1e-6
    lane = lax.broadcasted_iota(jnp.int32, nl.shape, 1)
    likeli = jnp.where(lane == 0, act_p / deno, noise_p / deno)
    slog = jnp.sum(jnp.log(likeli))

    pred = lax.dot_general(hp_ref[...], wt_ref[...], (((1,), (0,)), ((), ())),
                           preferred_element_type=jnp.float32)
    dt = pred / GRAN - (tt_ref[...] - lt_ref[...]) / GRAN
    ssq = jnp.sum(dt * dt)

    acc_ref[0] += slog
    acc_ref[1] += ssq

    @pl.when(i == pl.num_programs(0) - 1)
    def _():
        val = -acc_ref[0] / (B * (K + 1)) + acc_ref[1] / (B * 5.0)
        out_ref[...] = jnp.reshape(val, (1, 1))


def _epi_call(noise_logits, ap_raw, h_p, w_time, last_time, target_time):
    return pl.pallas_call(
        _epi_body,
        grid=(B // _BD,),
        in_specs=[
            pl.BlockSpec((_BD, K + 1), lambda i: (i, 0)),
            pl.BlockSpec((_BD, K + 1), lambda i: (i, 0)),
            pl.BlockSpec((_BD, D), lambda i: (i, 0)),
            pl.BlockSpec((D, 1), lambda i: (0, 0)),
            pl.BlockSpec((_BD, 1), lambda i: (i, 0)),
            pl.BlockSpec((_BD, 1), lambda i: (i, 0)),
        ],
        out_specs=pl.BlockSpec((1, 1), lambda i: (0, 0)),
        out_shape=jax.ShapeDtypeStruct((1, 1), jnp.float32),
        scratch_shapes=[pltpu.SMEM((2,), jnp.float32)],
    )(noise_logits, ap_raw, h_p, w_time, last_time, target_time)


def kernel(item_seq, item_seq_len, target_id, time_seq, time_interval_seq,
           target_time, q_emb, p_emb, q_time_w, p_time_w, w_time):
    gumbel = _gumbel_const()
    q_emb_pad = jnp.pad(q_emb, ((0, VP - V), (0, 0)))
    p_emb_pad = jnp.pad(p_emb, ((0, VP - V), (0, 0)))

    pad_ids = jnp.broadcast_to(jnp.arange(L, LP, dtype=jnp.int32)[None],
                               (B, LP - L))
    iseq_flat = jnp.concatenate([item_seq.astype(jnp.int32), pad_ids],
                                axis=1).reshape(-1)
    ts_flat = jnp.pad(time_seq, ((0, 0), (0, LP - L))).reshape(-1)
    ti_flat = jnp.pad(time_interval_seq, ((0, 0), (0, LP - L))).reshape(-1)

    q_emb_sc = jnp.pad(q_emb, ((0, 0), (0, 128 - D)))
    p_emb_sc = jnp.pad(p_emb, ((0, 0), (0, 128 - D)))
    pq_flat, pp_flat, smalls = _encode_sc_call(
        iseq_flat, ts_flat, ti_flat, item_seq_len.astype(jnp.int32),
        q_emb_sc, p_emb_sc)
    sm = smalls.reshape(B, 8)
    tm, im, last_time = sm[:, 0:1], sm[:, 1:2], sm[:, 2:3]
    tt = target_time.reshape(B, 1)

    scores, cm3, target_logit, ap0, h_p = _scores_call(
        pq_flat.reshape(B, D), pp_flat.reshape(B, D), tm, im, tt,
        q_time_w, p_time_w, q_emb_pad, p_emb_pad, gumbel,
        target_id.reshape(B, 1).astype(jnp.int32))

    cm_flat = jnp.pad(cm3.reshape(NCH, B).T, ((0, 0), (0, CMP - NCH)),
                      constant_values=NEG).reshape(-1)
    nl_flat, ap_flat = _select_call(cm_flat, scores.reshape(B * NCH, CH),
                                    gumbel.reshape(-1), h_p.reshape(-1),
                                    p_emb_sc)
    noise_logits = jnp.concatenate(
        [target_logit, nl_flat.reshape(B, GNE)[:, :K]], axis=1)
    ap_raw = jnp.concatenate([ap0, ap_flat.reshape(B, GNE)[:, :K]], axis=1)

    out = _epi_call(noise_logits, ap_raw, h_p, w_time, last_time, tt)
    return out.reshape(())
